# baseline (speedup 1.0000x reference)
import jax
import jax.numpy as jnp
from jax import lax
from jax.experimental import pallas as pl
from jax.experimental.pallas import tpu as pltpu
from jax.experimental.pallas import tpu_sc as plsc

_INFO = plsc.get_sparse_core_info()
_NC, _NS, _L = _INFO.num_cores, _INFO.num_subcores, _INFO.num_lanes
_NW = _NC * _NS

_D8 = 1024
_C = 32
_SUP = 1280
_NCH = _SUP // _C


def _sc_body(idx_hbm, tab_hbm, out_hbm,
             idxs, cidx, rows0, rows1, gsem0, gsem1, osem0, osem1):
    cid = lax.axis_index("c")
    sid = lax.axis_index("s")
    wid = sid * _NC + cid
    n_quads = idx_hbm.shape[1]
    per_w = n_quads // _NW
    n_super = per_w // _SUP

    rows = (rows0, rows1)
    gsems = (gsem0, gsem1)
    osems = (osem0, osem1)

    def super_body(s_i, _):
        sup_start = wid * per_w + s_i * _SUP
        sl = pl.ds(sup_start, _SUP)
        for k in range(8):
            pltpu.sync_copy(idx_hbm.at[k, sl], idxs.at[k])

        def cvt(i, _):
            s = pl.ds(i * _L, _L)
            p0 = (idxs[0, s] * 5 + idxs[1, s]) * 20 + idxs[2, s] * 5 + idxs[3, s]
            p1 = (idxs[4, s] * 5 + idxs[5, s]) * 20 + idxs[6, s] * 5 + idxs[7, s]
            cidx[s] = p0 * 400 + p1
            return _

        lax.fori_loop(0, _SUP // _L, cvt, 0)

        def pair_body(p, _):
            for b in range(2):
                ch = p * 2 + b
                first_use = (s_i == 0) & (p == 0)

                @pl.when(jnp.logical_not(first_use))
                def _wait():
                    pltpu.make_async_copy(
                        rows[b], out_hbm.at[pl.ds(0, _C)], osems[b]).wait()

                pltpu.async_copy(
                    tab_hbm.at[cidx.at[pl.ds(ch * _C, _C)]],
                    rows[b], gsems[b]).wait()
                out_off = sup_start + ch * _C
                pltpu.async_copy(
                    rows[b], out_hbm.at[pl.ds(out_off, _C)], osems[b])
            return _

        lax.fori_loop(0, _NCH // 2, pair_body, 0)
        return _

    lax.fori_loop(0, n_super, super_body, 0)

    for b in range(2):
        pltpu.make_async_copy(
            rows[b], out_hbm.at[pl.ds(0, _C)], osems[b]).wait()


@jax.jit
def kernel(batch_base_seq, batch_ref_seq, read_table, ref_table):
    B, S = batch_base_seq.shape
    D = read_table.shape[1]
    N = B * S
    c = jnp.arange(20)
    combined = jnp.concatenate(
        [read_table[c // 5], ref_table[c % 5]], axis=1)
    cp = jnp.arange(400)
    pair_tab = jnp.concatenate(
        [combined[cp // 20], combined[cp % 20]], axis=1)
    cq = jnp.arange(160000)
    quad_tab = jnp.concatenate(
        [pair_tab[cq // 400], pair_tab[cq % 400]], axis=1)

    base = batch_base_seq.astype(jnp.int32).reshape(N // 4, 4)
    refi = batch_ref_seq.astype(jnp.int32).reshape(N // 4, 4)
    idx8 = jnp.stack([
        base[:, 0], refi[:, 0], base[:, 1], refi[:, 1],
        base[:, 2], refi[:, 2], base[:, 3], refi[:, 3],
    ])

    run = pl.kernel(
        _sc_body,
        out_type=jax.ShapeDtypeStruct((N // 4, 8 * D), jnp.float32),
        mesh=plsc.VectorSubcoreMesh(core_axis_name="c", subcore_axis_name="s"),
        scratch_types=[
            pltpu.VMEM((8, _SUP), jnp.int32),
            pltpu.VMEM((_SUP,), jnp.int32),
            pltpu.VMEM((_C, _D8), jnp.float32),
            pltpu.VMEM((_C, _D8), jnp.float32),
            pltpu.SemaphoreType.DMA,
            pltpu.SemaphoreType.DMA,
            pltpu.SemaphoreType.DMA,
            pltpu.SemaphoreType.DMA,
        ],
    )
    out = run(idx8, quad_tab)
    return out.reshape(B, S, 2 * D)


# speedup vs baseline: 2.5562x; 2.5562x over previous
import functools

import jax
import jax.numpy as jnp
from jax.experimental import pallas as pl
from jax.experimental.pallas import tpu as pltpu

M = 8192


def _embed_body(base_ref, refi_ref, tab_ref, out_ref):
    cidx = base_ref[...] * 5 + refi_ref[...]
    iota = jax.lax.broadcasted_iota(jnp.int32, (M, 32), 1)
    onehot = (cidx == iota).astype(jnp.float32)
    out_ref[...] = jax.lax.dot_general(
        onehot, tab_ref[...],
        dimension_numbers=(((1,), (0,)), ((), ())),
        preferred_element_type=jnp.float32,
    )


@jax.jit
def kernel(batch_base_seq, batch_ref_seq, read_table, ref_table):
    B, S = batch_base_seq.shape
    D = read_table.shape[1]
    N = B * S
    c = jnp.arange(20)
    combined = jnp.concatenate(
        [read_table[c // 5], ref_table[c % 5]], axis=1)
    tab = jnp.pad(combined, ((0, 12), (0, 0)))
    base = batch_base_seq.astype(jnp.int32).reshape(N, 1)
    refi = batch_ref_seq.astype(jnp.int32).reshape(N, 1)

    out = pl.pallas_call(
        _embed_body,
        grid=(N // M,),
        in_specs=[
            pl.BlockSpec((M, 1), lambda i: (i, 0)),
            pl.BlockSpec((M, 1), lambda i: (i, 0)),
            pl.BlockSpec((32, 2 * D), lambda i: (0, 0)),
        ],
        out_specs=pl.BlockSpec((M, 2 * D), lambda i: (i, 0)),
        out_shape=jax.ShapeDtypeStruct((N, 2 * D), jnp.float32),
        compiler_params=pltpu.CompilerParams(
            dimension_semantics=("parallel",)),
    )(base, refi, tab)
    return out.reshape(B, S, 2 * D)
